# baseline (speedup 1.0000x reference)
import functools

import jax
import jax.numpy as jnp
from jax import lax
from jax.experimental import pallas as pl
from jax.experimental.pallas import tpu as pltpu
from jax.experimental.pallas import tpu_sc as plsc

BATCH = 16384
DIM = 128
NUM_CORES = 2
NUM_SUBCORES = 16
NW = NUM_CORES * NUM_SUBCORES
B_PER_W = BATCH // NW
CHUNK = 128
NCHUNK = B_PER_W // CHUNK


NBUF = 2


def _gather_kernel(idx_hbm, table_hbm, out_hbm, idx_v, *scr):
    rows = scr[:NBUF]
    gsems = scr[NBUF:2 * NBUF]
    wsems = scr[2 * NBUF:]
    wid = lax.axis_index("s") * NUM_CORES + lax.axis_index("c")
    base = wid * B_PER_W
    pltpu.sync_copy(idx_hbm.at[wid], idx_v)
    gathers = [None] * NCHUNK
    writes = [None] * NCHUNK
    for j in range(NBUF):
        gathers[j] = pltpu.async_copy(
            table_hbm.at[idx_v.at[j]], rows[j], gsems[j]
        )
    for j in range(NCHUNK):
        b = j % NBUF
        gathers[j].wait()
        writes[j] = pltpu.async_copy(
            rows[b], out_hbm.at[pl.ds(base + j * CHUNK, CHUNK)], wsems[b]
        )
        nxt = j + NBUF
        if nxt < NCHUNK:
            writes[j].wait()
            gathers[nxt] = pltpu.async_copy(
                table_hbm.at[idx_v.at[nxt]], rows[b], gsems[b]
            )
    for j in range(NCHUNK - NBUF, NCHUNK):
        if writes[j] is not None:
            writes[j].wait()


@jax.jit
def _run(indexes, entity_table):
    mesh = plsc.VectorSubcoreMesh(core_axis_name="c", subcore_axis_name="s")
    scratch = (
        [pltpu.VMEM((NCHUNK, CHUNK), jnp.int32)]
        + [pltpu.VMEM((CHUNK, DIM), jnp.float32) for _ in range(NBUF)]
        + [pltpu.SemaphoreType.DMA for _ in range(2 * NBUF)]
    )
    k = functools.partial(
        pl.kernel,
        mesh=mesh,
        out_type=jax.ShapeDtypeStruct((BATCH, DIM), jnp.float32),
        scratch_types=scratch,
    )(_gather_kernel)
    return k(indexes.reshape(NW, NCHUNK, CHUNK), entity_table)


def kernel(indexes, entity_table):
    return _run(indexes.astype(jnp.int32), entity_table)


# speedup vs baseline: 1.0388x; 1.0388x over previous
import functools

import jax
import jax.numpy as jnp
from jax import lax
from jax.experimental import pallas as pl
from jax.experimental.pallas import tpu as pltpu
from jax.experimental.pallas import tpu_sc as plsc

BATCH = 16384
DIM = 128
NUM_CORES = 2
NUM_SUBCORES = 16
NW = NUM_CORES * NUM_SUBCORES
B_PER_W = BATCH // NW
CHUNK = 128
NCHUNK = B_PER_W // CHUNK


def _gather_kernel(idx_hbm, table_hbm, out_hbm, idx_v, *scr):
    rows = scr[:NCHUNK]
    isems = scr[NCHUNK:2 * NCHUNK]
    gsems = scr[2 * NCHUNK:3 * NCHUNK]
    wsems = scr[3 * NCHUNK:]
    wid = lax.axis_index("s") * NUM_CORES + lax.axis_index("c")
    base = wid * B_PER_W
    icopies = [
        pltpu.async_copy(idx_hbm.at[wid, j], idx_v.at[j], isems[j])
        for j in range(NCHUNK)
    ]
    gathers = []
    for j in range(NCHUNK):
        icopies[j].wait()
        gathers.append(
            pltpu.async_copy(table_hbm.at[idx_v.at[j]], rows[j], gsems[j])
        )
    writes = []
    for j in range(NCHUNK):
        gathers[j].wait()
        writes.append(
            pltpu.async_copy(
                rows[j], out_hbm.at[pl.ds(base + j * CHUNK, CHUNK)], wsems[j]
            )
        )
    for w in writes:
        w.wait()


@jax.jit
def _run(indexes, entity_table):
    mesh = plsc.VectorSubcoreMesh(core_axis_name="c", subcore_axis_name="s")
    scratch = (
        [pltpu.VMEM((NCHUNK, CHUNK), jnp.int32)]
        + [pltpu.VMEM((CHUNK, DIM), jnp.float32) for _ in range(NCHUNK)]
        + [pltpu.SemaphoreType.DMA for _ in range(3 * NCHUNK)]
    )
    k = functools.partial(
        pl.kernel,
        mesh=mesh,
        out_type=jax.ShapeDtypeStruct((BATCH, DIM), jnp.float32),
        scratch_types=scratch,
    )(_gather_kernel)
    return k(indexes.reshape(NW, NCHUNK, CHUNK), entity_table)


def kernel(indexes, entity_table):
    return _run(indexes.astype(jnp.int32), entity_table)
